# Initial kernel scaffold; baseline (speedup 1.0000x reference)
import jax
import jax.numpy as jnp
from jax.experimental import pallas as pl


def kernel(pred_flows, kps):
    raise NotImplementedError("write your pallas kernel here")


# speedup vs baseline: 463.8493x; 463.8493x over previous
import functools

import jax
import jax.numpy as jnp
import numpy as np
from jax import lax
from jax.experimental import pallas as pl
from jax.experimental.pallas import tpu as pltpu
from jax.experimental.pallas import tpu_sc as plsc

L_LVL, B, C, H, W = 4, 16, 2, 256, 256
K = 128
HW = H * W
LANES = 16
HALF = K // 2
NCHUNK = HALF // LANES
NROW = L_LVL * C
NS = 16
GAMMAS = [float(0.8 ** (L_LVL - 1 - i)) for i in range(L_LVL)]

_mesh = plsc.VectorSubcoreMesh(core_axis_name="c", subcore_axis_name="s")


@functools.partial(
    pl.kernel,
    out_type=jax.ShapeDtypeStruct((2, NS, LANES), jnp.float32),
    mesh=_mesh,
    compiler_params=pltpu.CompilerParams(needs_layout_passes=False),
    scratch_types=[
        pltpu.VMEM((K,), jnp.int32),
        pltpu.VMEM((K,), jnp.int32),
        pltpu.VMEM((K,), jnp.int32),
        pltpu.VMEM((HALF,), jnp.int32),
        pltpu.VMEM((HALF,), jnp.int32),
        pltpu.VMEM((HALF,), jnp.int32),
        pltpu.VMEM((HALF,), jnp.int32),
        pltpu.VMEM((HALF,), jnp.int32),
        pltpu.VMEM((HW,), jnp.int32),
        pltpu.VMEM((NROW, HALF), jnp.int32),
        pltpu.VMEM((NROW, HALF), jnp.float32),
        pltpu.VMEM((LANES,), jnp.float32),
        pltpu.SemaphoreType.DMA,
    ],
)
def _loss_sc(x0_h, y0_h, x1_h, y1_h, pred_h, out_h,
             x0_v, y0_v, pos_v, mx0_v, my0_v, mx1_v, my1_v, mpos_v,
             table, idx_v, vals_v, part_v, sem):
    c = lax.axis_index("c")
    s = lax.axis_index("s")
    b = s
    base_k = c * HALF

    pltpu.sync_copy(x0_h.at[b], x0_v)
    pltpu.sync_copy(y0_h.at[b], y0_v)
    pltpu.sync_copy(x0_h.at[b, pl.ds(base_k, HALF)], mx0_v)
    pltpu.sync_copy(y0_h.at[b, pl.ds(base_k, HALF)], my0_v)
    pltpu.sync_copy(x1_h.at[b, pl.ds(base_k, HALF)], mx1_v)
    pltpu.sync_copy(y1_h.at[b, pl.ds(base_k, HALF)], my1_v)

    for j in range(K // LANES):
        sl = pl.ds(j * LANES, LANES)
        pos_v[sl] = y0_v[sl] * W + x0_v[sl]
    for j in range(NCHUNK):
        sl = pl.ds(j * LANES, LANES)
        mpos_v[sl] = my0_v[sl] * W + mx0_v[sl]

    lane = lax.iota(jnp.int32, LANES)
    for j in range(K // LANES):
        sl = pl.ds(j * LANES, LANES)
        plsc.store_scatter(table, [pos_v[sl]], j * LANES + lane)

    for j in range(NCHUNK):
        sl = pl.ds(j * LANES, LANES)
        p = mpos_v[sl]
        for i in range(L_LVL):
            for cc in range(C):
                r = i * C + cc
                off = (i * B * C + cc) * HW + b * (C * HW)
                idx_v[r, sl] = p + off

    copies = [pltpu.async_copy(pred_h.at[idx_v.at[r]], vals_v.at[r], sem)
              for r in range(NROW)]
    for cp in copies:
        cp.wait()

    acc = jnp.zeros((LANES,), jnp.float32)
    cnt_acc = jnp.zeros((LANES,), jnp.float32)
    for j in range(NCHUNK):
        sl = pl.ds(j * LANES, LANES)
        p = mpos_v[sl]
        winner = plsc.load_gather(table, [p]) == (base_k + j * LANES + lane)
        x0c = mx0_v[sl]
        y0c = my0_v[sl]
        x1c = mx1_v[sl]
        y1c = my1_v[sl]
        nz = (x1c != x0c) | (y1c != y0c)
        w = jnp.where(winner & nz, 1.0, 0.0).astype(jnp.float32)
        cnt_acc = cnt_acc + w
        dxf = (x1c - x0c).astype(jnp.float32)
        dyf = (y1c - y0c).astype(jnp.float32)
        for i in range(L_LVL):
            d0 = vals_v[2 * i, sl] - dxf
            d1 = vals_v[2 * i + 1, sl] - dyf
            acc = acc + (np.float32(GAMMAS[i]) * w) * (d0 * d0 + d1 * d1)

    ws = jnp.sum(acc)
    cnt = jnp.sum(cnt_acc)
    part_v[...] = jnp.where(lane == 0, ws, jnp.where(lane == 1, cnt, 0.0))
    pltpu.sync_copy(part_v, out_h.at[c, s])


def kernel(pred_flows, kps):
    x0 = kps[:, 0, :, 0]
    y0 = kps[:, 0, :, 1]
    x1 = kps[:, 1, :, 0]
    y1 = kps[:, 1, :, 1]
    pred_flat = pred_flows.reshape(-1)
    parts = _loss_sc(x0, y0, x1, y1, pred_flat)
    tot = jnp.sum(parts, axis=(0, 1))
    return tot[0] / (2.0 * tot[1])


# speedup vs baseline: 480.1443x; 1.0351x over previous
import functools

import jax
import jax.numpy as jnp
import numpy as np
from jax import lax
from jax.experimental import pallas as pl
from jax.experimental.pallas import tpu as pltpu
from jax.experimental.pallas import tpu_sc as plsc

L_LVL, B, C, H, W = 4, 16, 2, 256, 256
K = 128
HW = H * W
LANES = 16
HALF = K // 2
NCHUNK = HALF // LANES
NROW = L_LVL * C
NS = 16
GAMMAS = [float(0.8 ** (L_LVL - 1 - i)) for i in range(L_LVL)]

_mesh = plsc.VectorSubcoreMesh(core_axis_name="c", subcore_axis_name="s")


@functools.partial(
    pl.kernel,
    out_type=jax.ShapeDtypeStruct((2, NS, LANES), jnp.float32),
    mesh=_mesh,
    compiler_params=pltpu.CompilerParams(needs_layout_passes=False),
    scratch_types=[
        pltpu.VMEM((K,), jnp.int32),
        pltpu.VMEM((K,), jnp.int32),
        pltpu.VMEM((HALF,), jnp.int32),
        pltpu.VMEM((HALF,), jnp.int32),
        pltpu.VMEM((HW,), jnp.int32),
        pltpu.VMEM((NROW, HALF), jnp.int32),
        pltpu.VMEM((NROW, HALF), jnp.float32),
        pltpu.VMEM((LANES,), jnp.float32),
        pltpu.SemaphoreType.DMA,
    ],
)
def _loss_sc(pk_h, pred_h, out_h,
             pk_v, pos_v, mpk_v, mpos_v,
             table, idx_v, vals_v, part_v, sem):
    c = lax.axis_index("c")
    s = lax.axis_index("s")
    b = s
    base_k = c * HALF

    pltpu.sync_copy(pk_h.at[b], pk_v)
    pltpu.sync_copy(pk_h.at[b, pl.ds(base_k, HALF)], mpk_v)

    for j in range(K // LANES):
        sl = pl.ds(j * LANES, LANES)
        pk = pk_v[sl]
        pos_v[sl] = ((pk >> 8) & 0xFF) * W + (pk & 0xFF)
    for j in range(NCHUNK):
        sl = pl.ds(j * LANES, LANES)
        pk = mpk_v[sl]
        mpos_v[sl] = ((pk >> 8) & 0xFF) * W + (pk & 0xFF)

    lane = lax.iota(jnp.int32, LANES)
    for j in range(K // LANES):
        sl = pl.ds(j * LANES, LANES)
        plsc.store_scatter(table, [pos_v[sl]], j * LANES + lane)

    for j in range(NCHUNK):
        sl = pl.ds(j * LANES, LANES)
        p = mpos_v[sl]
        for i in range(L_LVL):
            for cc in range(C):
                r = i * C + cc
                off = (i * B * C + cc) * HW + b * (C * HW)
                idx_v[r, sl] = p + off

    copies = [pltpu.async_copy(pred_h.at[idx_v.at[r]], vals_v.at[r], sem)
              for r in range(NROW)]
    for cp in copies:
        cp.wait()

    acc = jnp.zeros((LANES,), jnp.float32)
    cnt_acc = jnp.zeros((LANES,), jnp.float32)
    for j in range(NCHUNK):
        sl = pl.ds(j * LANES, LANES)
        p = mpos_v[sl]
        winner = plsc.load_gather(table, [p]) == (base_k + j * LANES + lane)
        pk = mpk_v[sl]
        x0c = pk & 0xFF
        y0c = (pk >> 8) & 0xFF
        x1c = (pk >> 16) & 0xFF
        y1c = (pk >> 24) & 0xFF
        nz = (x1c != x0c) | (y1c != y0c)
        w = jnp.where(winner & nz, 1.0, 0.0).astype(jnp.float32)
        cnt_acc = cnt_acc + w
        dxf = (x1c - x0c).astype(jnp.float32)
        dyf = (y1c - y0c).astype(jnp.float32)
        for i in range(L_LVL):
            d0 = vals_v[2 * i, sl] - dxf
            d1 = vals_v[2 * i + 1, sl] - dyf
            acc = acc + (np.float32(GAMMAS[i]) * w) * (d0 * d0 + d1 * d1)

    ws = jnp.sum(acc)
    cnt = jnp.sum(cnt_acc)
    part_v[...] = jnp.where(lane == 0, ws, jnp.where(lane == 1, cnt, 0.0))
    pltpu.sync_copy(part_v, out_h.at[c, s])


def kernel(pred_flows, kps):
    packed = (kps[:, 0, :, 0]
              | (kps[:, 0, :, 1] << 8)
              | (kps[:, 1, :, 0] << 16)
              | (kps[:, 1, :, 1] << 24))
    pred_flat = pred_flows.reshape(-1)
    parts = _loss_sc(packed, pred_flat)
    tot = jnp.sum(parts, axis=(0, 1))
    return tot[0] / (2.0 * tot[1])


# speedup vs baseline: 1015.5069x; 2.1150x over previous
import functools

import jax
import jax.numpy as jnp
import numpy as np
from jax import lax
from jax.experimental import pallas as pl
from jax.experimental.pallas import tpu as pltpu
from jax.experimental.pallas import tpu_sc as plsc

L_LVL, B, C, H, W = 4, 16, 2, 256, 256
K = 128
HW = H * W
LANES = 16
HALF = K // 2
NCHUNK = HALF // LANES
NROW = L_LVL * C
NS = 16
GAMMAS = [float(0.8 ** (L_LVL - 1 - i)) for i in range(L_LVL)]

_mesh = plsc.VectorSubcoreMesh(core_axis_name="c", subcore_axis_name="s")


@functools.partial(
    pl.kernel,
    out_type=jax.ShapeDtypeStruct((2, NS, LANES), jnp.float32),
    mesh=_mesh,
    compiler_params=pltpu.CompilerParams(needs_layout_passes=False),
    scratch_types=[
        pltpu.VMEM((K,), jnp.int32),
        pltpu.VMEM((K,), jnp.int32),
        pltpu.VMEM((HALF,), jnp.int32),
        pltpu.VMEM((HALF,), jnp.int32),
        pltpu.VMEM((HW,), jnp.int32),
        pltpu.VMEM((NROW, HALF), jnp.int32),
        pltpu.VMEM((NROW, HALF), jnp.float32),
        pltpu.VMEM((LANES,), jnp.float32),
        pltpu.SemaphoreType.DMA,
    ],
)
def _loss_sc(pk_h, pred_h, out_h,
             pk_v, pos_v, mpk_v, mpos_v,
             table, idx_v, vals_v, part_v, sem):
    c = lax.axis_index("c")
    s = lax.axis_index("s")
    b = s
    base_k = c * HALF

    pltpu.sync_copy(pk_h.at[b], pk_v)
    pltpu.sync_copy(pk_h.at[b, pl.ds(base_k, HALF)], mpk_v)

    for j in range(K // LANES):
        sl = pl.ds(j * LANES, LANES)
        pk = pk_v[sl]
        pos_v[sl] = ((pk >> 8) & 0xFF) * W + (pk & 0xFF)
    for j in range(NCHUNK):
        sl = pl.ds(j * LANES, LANES)
        pk = mpk_v[sl]
        mpos_v[sl] = ((pk >> 8) & 0xFF) * W + (pk & 0xFF)

    lane = lax.iota(jnp.int32, LANES)
    for j in range(K // LANES):
        sl = pl.ds(j * LANES, LANES)
        plsc.store_scatter(table, [pos_v[sl]], j * LANES + lane)

    for j in range(NCHUNK):
        sl = pl.ds(j * LANES, LANES)
        p = mpos_v[sl]
        tp = (((p >> 11) << 11) | (((p >> 7) & 1) << 10)
              | (((p >> 8) & 7) << 7) | (p & 127))
        for i in range(L_LVL):
            for cc in range(C):
                r = i * C + cc
                off = (i * B * C + cc) * HW + b * (C * HW)
                idx_v[r, sl] = tp + off

    copies = [pltpu.async_copy(pred_h.at[idx_v.at[r]], vals_v.at[r], sem)
              for r in range(NROW)]
    for cp in copies:
        cp.wait()

    acc = jnp.zeros((LANES,), jnp.float32)
    cnt_acc = jnp.zeros((LANES,), jnp.float32)
    for j in range(NCHUNK):
        sl = pl.ds(j * LANES, LANES)
        p = mpos_v[sl]
        winner = plsc.load_gather(table, [p]) == (base_k + j * LANES + lane)
        pk = mpk_v[sl]
        x0c = pk & 0xFF
        y0c = (pk >> 8) & 0xFF
        x1c = (pk >> 16) & 0xFF
        y1c = (pk >> 24) & 0xFF
        nz = (x1c != x0c) | (y1c != y0c)
        w = jnp.where(winner & nz, 1.0, 0.0).astype(jnp.float32)
        cnt_acc = cnt_acc + w
        dxf = (x1c - x0c).astype(jnp.float32)
        dyf = (y1c - y0c).astype(jnp.float32)
        for i in range(L_LVL):
            d0 = vals_v[2 * i, sl] - dxf
            d1 = vals_v[2 * i + 1, sl] - dyf
            acc = acc + (np.float32(GAMMAS[i]) * w) * (d0 * d0 + d1 * d1)

    ws = jnp.sum(acc)
    cnt = jnp.sum(cnt_acc)
    part_v[...] = jnp.where(lane == 0, ws, jnp.where(lane == 1, cnt, 0.0))
    pltpu.sync_copy(part_v, out_h.at[c, s])


def kernel(pred_flows, kps):
    packed = (kps[:, 0, :, 0]
              | (kps[:, 0, :, 1] << 8)
              | (kps[:, 1, :, 0] << 16)
              | (kps[:, 1, :, 1] << 24))
    pred_flat = (pred_flows
                 .reshape(L_LVL, B, C, H // 8, 8, W // 128, 128)
                 .transpose(0, 1, 2, 3, 5, 4, 6)
                 .reshape(-1))
    parts = _loss_sc(packed, pred_flat)
    tot = jnp.sum(parts, axis=(0, 1))
    return tot[0] / (2.0 * tot[1])


# speedup vs baseline: 1019.0209x; 1.0035x over previous
import functools

import jax
import jax.numpy as jnp
import numpy as np
from jax import lax
from jax.experimental import pallas as pl
from jax.experimental.pallas import tpu as pltpu
from jax.experimental.pallas import tpu_sc as plsc

L_LVL, B, C, H, W = 4, 16, 2, 256, 256
K = 128
HW = H * W
LANES = 16
HALF = K // 2
NCHUNK = HALF // LANES
NROW = L_LVL * C
NS = 16
GAMMAS = [float(0.8 ** (L_LVL - 1 - i)) for i in range(L_LVL)]

_mesh = plsc.VectorSubcoreMesh(core_axis_name="c", subcore_axis_name="s")


@functools.partial(
    pl.kernel,
    out_type=jax.ShapeDtypeStruct((2, NS, LANES), jnp.float32),
    mesh=_mesh,
    compiler_params=pltpu.CompilerParams(needs_layout_passes=False),
    scratch_types=[
        pltpu.VMEM((K,), jnp.int32),
        pltpu.VMEM((K,), jnp.int32),
        pltpu.VMEM((HALF,), jnp.int32),
        pltpu.VMEM((HALF,), jnp.int32),
        pltpu.VMEM((HW,), jnp.int32),
        pltpu.VMEM((NROW * HALF // 128, 128), jnp.int32),
        pltpu.VMEM((NROW * HALF // 128, 128), jnp.float32),
        pltpu.VMEM((LANES,), jnp.float32),
        pltpu.SemaphoreType.DMA,
    ],
)
def _loss_sc(pk_h, pred_h, out_h,
             pk_v, pos_v, mpk_v, mpos_v,
             table, idx_v, vals_v, part_v, sem):
    c = lax.axis_index("c")
    s = lax.axis_index("s")
    b = s
    base_k = c * HALF

    pltpu.sync_copy(pk_h.at[b], pk_v)
    pltpu.sync_copy(pk_h.at[b, pl.ds(base_k, HALF)], mpk_v)

    for j in range(K // LANES):
        sl = pl.ds(j * LANES, LANES)
        pk = pk_v[sl]
        pos_v[sl] = ((pk >> 8) & 0xFF) * W + (pk & 0xFF)
    for j in range(NCHUNK):
        sl = pl.ds(j * LANES, LANES)
        pk = mpk_v[sl]
        mpos_v[sl] = ((pk >> 8) & 0xFF) * W + (pk & 0xFF)

    lane = lax.iota(jnp.int32, LANES)
    for j in range(K // LANES):
        sl = pl.ds(j * LANES, LANES)
        plsc.store_scatter(table, [pos_v[sl]], j * LANES + lane)

    for j in range(NCHUNK):
        sl = pl.ds(j * LANES, LANES)
        p = mpos_v[sl]
        tp = (((p >> 11) << 11) | (((p >> 7) & 1) << 10)
              | (((p >> 8) & 7) << 7) | (p & 127))
        for i in range(L_LVL):
            for cc in range(C):
                r = i * C + cc
                off = (i * B * C + cc) * HW + b * (C * HW)
                e0 = r * HALF + j * LANES
                idx_v[e0 >> 7, pl.ds(e0 & 127, LANES)] = tp + off

    copies = [pltpu.async_copy(pred_h.at[idx_v.at[r]], vals_v.at[r], sem)
              for r in range(NROW * HALF // 128)]
    for cp in copies:
        cp.wait()

    acc = jnp.zeros((LANES,), jnp.float32)
    cnt_acc = jnp.zeros((LANES,), jnp.float32)
    for j in range(NCHUNK):
        sl = pl.ds(j * LANES, LANES)
        p = mpos_v[sl]
        winner = plsc.load_gather(table, [p]) == (base_k + j * LANES + lane)
        pk = mpk_v[sl]
        x0c = pk & 0xFF
        y0c = (pk >> 8) & 0xFF
        x1c = (pk >> 16) & 0xFF
        y1c = (pk >> 24) & 0xFF
        nz = (x1c != x0c) | (y1c != y0c)
        w = jnp.where(winner & nz, 1.0, 0.0).astype(jnp.float32)
        cnt_acc = cnt_acc + w
        dxf = (x1c - x0c).astype(jnp.float32)
        dyf = (y1c - y0c).astype(jnp.float32)
        for i in range(L_LVL):
            e0 = (2 * i) * HALF + j * LANES
            e1 = (2 * i + 1) * HALF + j * LANES
            d0 = vals_v[e0 >> 7, pl.ds(e0 & 127, LANES)] - dxf
            d1 = vals_v[e1 >> 7, pl.ds(e1 & 127, LANES)] - dyf
            acc = acc + (np.float32(GAMMAS[i]) * w) * (d0 * d0 + d1 * d1)

    ws = jnp.sum(acc)
    cnt = jnp.sum(cnt_acc)
    part_v[...] = jnp.where(lane == 0, ws, jnp.where(lane == 1, cnt, 0.0))
    pltpu.sync_copy(part_v, out_h.at[c, s])


def kernel(pred_flows, kps):
    packed = (kps[:, 0, :, 0]
              | (kps[:, 0, :, 1] << 8)
              | (kps[:, 1, :, 0] << 16)
              | (kps[:, 1, :, 1] << 24))
    pred_flat = (pred_flows
                 .reshape(L_LVL, B, C, H // 8, 8, W // 128, 128)
                 .transpose(0, 1, 2, 3, 5, 4, 6)
                 .reshape(-1))
    parts = _loss_sc(packed, pred_flat)
    tot = jnp.sum(parts, axis=(0, 1))
    return tot[0] / (2.0 * tot[1])
